# Initial kernel scaffold; baseline (speedup 1.0000x reference)
import jax
import jax.numpy as jnp
from jax.experimental import pallas as pl


def kernel(sparse_repr):
    raise NotImplementedError("write your pallas kernel here")


# speedup vs baseline: 1.0854x; 1.0854x over previous
import jax
import jax.numpy as jnp
from jax import lax
from jax.experimental import pallas as pl
from jax.experimental.pallas import tpu as pltpu
from jax.experimental.pallas import tpu_sc as plsc

_R = 1024
_N = 100000
_L = 16
_NC = 2
_NS = 16
_NW = _NC * _NS
_RPW = _R // _NW
_NV = _N // _L
_CAND = 128
_TOP_K = 5
_MIN_ACT = 0.5
_NEG = float(-jnp.inf)


def _sc_body(x_hbm, cand_hbm, row_buf, out_buf):
    cid = lax.axis_index("c")
    sid = lax.axis_index("s")
    wid = cid * _NS + sid
    neg = jnp.full((_L,), _NEG, jnp.float32)

    for j in range(5, 8):
        out_buf[pl.ds(j * _L, _L)] = neg

    def row_step(r, carry):
        row = wid * _RPW + r
        pltpu.sync_copy(x_hbm.at[row], row_buf)

        def vec_step(i, tops):
            t0, t1, t2, t3, t4 = tops
            x = row_buf[pl.ds(i * _L, _L)]
            m0 = jnp.maximum(t0, x)
            c = jnp.minimum(t0, x)
            m1 = jnp.maximum(t1, c)
            c = jnp.minimum(t1, c)
            m2 = jnp.maximum(t2, c)
            c = jnp.minimum(t2, c)
            m3 = jnp.maximum(t3, c)
            c = jnp.minimum(t3, c)
            m4 = jnp.maximum(t4, c)
            return (m0, m1, m2, m3, m4)

        tops = lax.fori_loop(0, _NV, vec_step, (neg, neg, neg, neg, neg))
        for j in range(5):
            out_buf[pl.ds(j * _L, _L)] = tops[j]
        pltpu.sync_copy(out_buf, cand_hbm.at[row])
        return carry

    lax.fori_loop(0, _RPW, row_step, jnp.int32(0))


def _tc_body(cand_ref, out_ref):
    x = cand_ref[...]
    col = lax.broadcasted_iota(jnp.int32, x.shape, 1)
    s = jnp.zeros((x.shape[0], 1), jnp.float32)
    for _ in range(_TOP_K):
        m = jnp.max(x, axis=1, keepdims=True)
        idx = jnp.min(jnp.where(x == m, col, jnp.int32(_CAND)),
                      axis=1, keepdims=True)
        x = jnp.where(col == idx, jnp.float32(_NEG), x)
        s = s + m
    loss = jnp.maximum(jnp.float32(_MIN_ACT) - s * jnp.float32(1.0 / _TOP_K),
                       jnp.float32(0.0))
    out_ref[...] = jnp.reshape(jnp.sum(loss) * jnp.float32(1.0 / _R), (1, 1))


@jax.jit
def kernel(sparse_repr):
    mesh = plsc.VectorSubcoreMesh(core_axis_name="c", subcore_axis_name="s")
    cand = pl.kernel(
        _sc_body,
        out_type=jax.ShapeDtypeStruct((_R, _CAND), jnp.float32),
        mesh=mesh,
        scratch_types=[
            pltpu.VMEM((_N,), jnp.float32),
            pltpu.VMEM((_CAND,), jnp.float32),
        ],
    )(sparse_repr)
    loss = pl.pallas_call(
        _tc_body,
        out_shape=jax.ShapeDtypeStruct((1, 1), jnp.float32),
        in_specs=[pl.BlockSpec(memory_space=pltpu.VMEM)],
        out_specs=pl.BlockSpec(memory_space=pltpu.VMEM),
    )(cand)
    return loss[0, 0]


# speedup vs baseline: 1.0870x; 1.0014x over previous
import jax
import jax.numpy as jnp
from jax import lax
from jax.experimental import pallas as pl
from jax.experimental.pallas import tpu as pltpu
from jax.experimental.pallas import tpu_sc as plsc

_R = 1024
_N = 100000
_L = 16
_NC = 2
_NS = 16
_NW = _NC * _NS
_RPW = _R // _NW
_NV = _N // _L
_CAND = 128
_TOP_K = 5
_MIN_ACT = 0.5
_NEG = float(-jnp.inf)


_NH = _N // 2
_NVH = _NH // _L


def _sc_body(x_hbm, cand_hbm, row_buf, out_buf):
    cid = lax.axis_index("c")
    sid = lax.axis_index("s")
    wid = cid * _NS + sid
    neg = jnp.full((_L,), _NEG, jnp.float32)

    for j in range(5, 8):
        out_buf[pl.ds(j * _L, _L)] = neg

    def row_step(r, carry):
        row = wid * _RPW + r
        pltpu.sync_copy(x_hbm.at[row], row_buf)

        def vec_step(i, tops):
            t0, t1, t2, t3, t4 = tops
            x = row_buf[pl.ds(i * _L, _L)]
            m0 = jnp.maximum(t0, x)
            c = jnp.minimum(t0, x)
            m1 = jnp.maximum(t1, c)
            c = jnp.minimum(t1, c)
            m2 = jnp.maximum(t2, c)
            c = jnp.minimum(t2, c)
            m3 = jnp.maximum(t3, c)
            c = jnp.minimum(t3, c)
            m4 = jnp.maximum(t4, c)
            return (m0, m1, m2, m3, m4)

        tops = plsc.parallel_loop(
            0, _NV, unroll=8, carry=(neg, neg, neg, neg, neg))(vec_step)
        for j in range(5):
            out_buf[pl.ds(j * _L, _L)] = tops[j]
        pltpu.sync_copy(out_buf, cand_hbm.at[row])
        return carry

    lax.fori_loop(0, _RPW, row_step, jnp.int32(0))


def _tc_body(cand_ref, out_ref):
    x = cand_ref[...]
    col = lax.broadcasted_iota(jnp.int32, x.shape, 1)
    s = jnp.zeros((x.shape[0], 1), jnp.float32)
    for _ in range(_TOP_K):
        m = jnp.max(x, axis=1, keepdims=True)
        idx = jnp.min(jnp.where(x == m, col, jnp.int32(_CAND)),
                      axis=1, keepdims=True)
        x = jnp.where(col == idx, jnp.float32(_NEG), x)
        s = s + m
    loss = jnp.maximum(jnp.float32(_MIN_ACT) - s * jnp.float32(1.0 / _TOP_K),
                       jnp.float32(0.0))
    out_ref[...] = jnp.reshape(jnp.sum(loss) * jnp.float32(1.0 / _R), (1, 1))


@jax.jit
def kernel(sparse_repr):
    mesh = plsc.VectorSubcoreMesh(core_axis_name="c", subcore_axis_name="s")
    cand = pl.kernel(
        _sc_body,
        out_type=jax.ShapeDtypeStruct((_R, _CAND), jnp.float32),
        mesh=mesh,
        scratch_types=[
            pltpu.VMEM((_N,), jnp.float32),
            pltpu.VMEM((_CAND,), jnp.float32),
        ],
    )(sparse_repr)
    loss = pl.pallas_call(
        _tc_body,
        out_shape=jax.ShapeDtypeStruct((1, 1), jnp.float32),
        in_specs=[pl.BlockSpec(memory_space=pltpu.VMEM)],
        out_specs=pl.BlockSpec(memory_space=pltpu.VMEM),
    )(cand)
    return loss[0, 0]


# speedup vs baseline: 1.6583x; 1.5256x over previous
import jax
import jax.numpy as jnp
from jax import lax
from jax.experimental import pallas as pl
from jax.experimental.pallas import tpu as pltpu
from jax.experimental.pallas import tpu_sc as plsc

_R = 1024
_N = 100000
_L = 16
_NC = 2
_NS = 16
_NW = _NC * _NS
_GPW = 4
_CW = 3200
_NCH = 31
_MAIN = _NCH * _CW
_TW = 1024
_CVEC = _CW // _L
_TVEC = _TW // _L
_CAND = 128
_TOP_K = 5
_MIN_ACT = 0.5
_NEG = float(-jnp.inf)


def _sc_body(x_hbm, tail_hbm, cand_hbm, cb, tb, tops_buf, out_buf,
             sem0, sem1, semt):
    cid = lax.axis_index("c")
    sid = lax.axis_index("s")
    wid = cid * _NS + sid
    neg = jnp.full((_L,), _NEG, jnp.float32)
    sems = (sem0, sem1)

    def chunk_copy(grp, c, par, sem):
        return pltpu.make_async_copy(
            x_hbm.at[pl.ds(grp * 8, 8), pl.ds(c * _CW, _CW)],
            cb.at[par], sem)

    def scan_pair(buf, par, i, nvec, tops):
        a0, a1, a2, a3, a4, b0, b1, b2, b3, b4 = tops

        def vec_step(j, t):
            a0, a1, a2, a3, a4, b0, b1, b2, b3, b4 = t
            x = buf[par, i, pl.ds(j * _L, _L)]
            y = buf[par, i + 1, pl.ds(j * _L, _L)]
            n0 = jnp.maximum(a0, x)
            p0 = jnp.maximum(b0, y)
            cx = jnp.minimum(a0, x)
            cy = jnp.minimum(b0, y)
            n1 = jnp.maximum(a1, cx)
            p1 = jnp.maximum(b1, cy)
            cx = jnp.minimum(a1, cx)
            cy = jnp.minimum(b1, cy)
            n2 = jnp.maximum(a2, cx)
            p2 = jnp.maximum(b2, cy)
            cx = jnp.minimum(a2, cx)
            cy = jnp.minimum(b2, cy)
            n3 = jnp.maximum(a3, cx)
            p3 = jnp.maximum(b3, cy)
            cx = jnp.minimum(a3, cx)
            cy = jnp.minimum(b3, cy)
            n4 = jnp.maximum(a4, cx)
            p4 = jnp.maximum(b4, cy)
            return (n0, n1, n2, n3, n4, p0, p1, p2, p3, p4)

        return plsc.parallel_loop(
            0, nvec, unroll=4,
            carry=(a0, a1, a2, a3, a4, b0, b1, b2, b3, b4))(vec_step)

    def load_tops(i):
        return tuple(tops_buf[i, k] for k in range(5)) + tuple(
            tops_buf[i + 1, k] for k in range(5))

    def store_tops(i, tops):
        for k in range(5):
            tops_buf[i, k] = tops[k]
            tops_buf[i + 1, k] = tops[5 + k]

    def group_step(g, carry):
        grp = wid * _GPW + g

        for i in range(8):
            for k in range(5):
                tops_buf[i, k] = neg
        chunk_copy(grp, 0, 0, sem0).start()
        pltpu.make_async_copy(tail_hbm.at[pl.ds(grp * 8, 8)], tb.at[0], semt).start()

        def chunk_step(c, carry):
            par = lax.rem(c, 2)

            @pl.when(par == 0)
            def _():
                chunk_copy(grp, c, 0, sem0).wait()

            @pl.when(par == 1)
            def _():
                chunk_copy(grp, c, 1, sem1).wait()

            npar = lax.rem(c + 1, 2)

            @pl.when((c + 1 < _NCH) & (npar == 0))
            def _():
                chunk_copy(grp, c + 1, 0, sem0).start()

            @pl.when((c + 1 < _NCH) & (npar == 1))
            def _():
                chunk_copy(grp, c + 1, 1, sem1).start()

            for i in range(0, 8, 2):
                tops = load_tops(i)
                tops = scan_pair(cb, par, i, _CVEC, tops)
                store_tops(i, tops)
            return carry

        lax.fori_loop(0, _NCH, chunk_step, jnp.int32(0))

        pltpu.make_async_copy(tail_hbm.at[pl.ds(grp * 8, 8)], tb.at[0], semt).wait()
        for i in range(0, 8, 2):
            tops = load_tops(i)
            tops = scan_pair(tb, 0, i, _TVEC, tops)
            for k in range(5):
                out_buf[i, pl.ds(k * _L, _L)] = tops[k]
                out_buf[i + 1, pl.ds(k * _L, _L)] = tops[5 + k]
            for k in range(5, 8):
                out_buf[i, pl.ds(k * _L, _L)] = neg
                out_buf[i + 1, pl.ds(k * _L, _L)] = neg
        pltpu.sync_copy(out_buf, cand_hbm.at[pl.ds(grp * 8, 8)])
        return carry

    lax.fori_loop(0, _GPW, group_step, jnp.int32(0))


def _tc_body(cand_ref, out_ref):
    x = cand_ref[...]
    col = lax.broadcasted_iota(jnp.int32, x.shape, 1)
    s = jnp.zeros((x.shape[0], 1), jnp.float32)
    for _ in range(_TOP_K):
        m = jnp.max(x, axis=1, keepdims=True)
        idx = jnp.min(jnp.where(x == m, col, jnp.int32(_CAND)),
                      axis=1, keepdims=True)
        x = jnp.where(col == idx, jnp.float32(_NEG), x)
        s = s + m
    loss = jnp.maximum(jnp.float32(_MIN_ACT) - s * jnp.float32(1.0 / _TOP_K),
                       jnp.float32(0.0))
    out_ref[...] = jnp.reshape(jnp.sum(loss) * jnp.float32(1.0 / _R), (1, 1))


@jax.jit
def kernel(sparse_repr):
    tail = jnp.concatenate(
        [sparse_repr[:, _MAIN:],
         jnp.full((_R, _TW - (_N - _MAIN)), _NEG, jnp.float32)], axis=1)
    mesh = plsc.VectorSubcoreMesh(core_axis_name="c", subcore_axis_name="s")
    cand = pl.kernel(
        _sc_body,
        out_type=jax.ShapeDtypeStruct((_R, _CAND), jnp.float32),
        mesh=mesh,
        scratch_types=[
            pltpu.VMEM((2, 8, _CW), jnp.float32),
            pltpu.VMEM((1, 8, _TW), jnp.float32),
            pltpu.VMEM((8, 5, _L), jnp.float32),
            pltpu.VMEM((8, _CAND), jnp.float32),
            pltpu.SemaphoreType.DMA,
            pltpu.SemaphoreType.DMA,
            pltpu.SemaphoreType.DMA,
        ],
    )(sparse_repr, tail)
    loss = pl.pallas_call(
        _tc_body,
        out_shape=jax.ShapeDtypeStruct((1, 1), jnp.float32),
        in_specs=[pl.BlockSpec(memory_space=pltpu.VMEM)],
        out_specs=pl.BlockSpec(memory_space=pltpu.VMEM),
    )(cand)
    return loss[0, 0]


# speedup vs baseline: 1.7191x; 1.0366x over previous
import jax
import jax.numpy as jnp
from jax import lax
from jax.experimental import pallas as pl
from jax.experimental.pallas import tpu as pltpu
from jax.experimental.pallas import tpu_sc as plsc

_R = 1024
_N = 100000
_L = 16
_NC = 2
_NS = 16
_NW = _NC * _NS
_GPW = 4
_CW = 3200
_NCH = 31
_MAIN = _NCH * _CW
_TW = 1024
_CVEC = _CW // _L
_TVEC = _TW // _L
_CAND = 128
_TOP_K = 5
_MIN_ACT = 0.5
_NEG = float(-jnp.inf)


def _sc_body(x_hbm, tail_hbm, cand_hbm, cb, tb, tops_buf, out_buf,
             sem0, sem1, semt):
    cid = lax.axis_index("c")
    sid = lax.axis_index("s")
    wid = cid * _NS + sid
    neg = jnp.full((_L,), _NEG, jnp.float32)
    sems = (sem0, sem1)

    def chunk_copy(grp, c, par, sem):
        return pltpu.make_async_copy(
            x_hbm.at[pl.ds(grp * 8, 8), pl.ds(c * _CW, _CW)],
            cb.at[par], sem)

    def scan_quad(buf, par, i, nvec, tops):
        def vec_step(j, t):
            t = list(t)
            xs = [buf[par, i + q, pl.ds(j * _L, _L)] for q in range(4)]
            for lev in range(5):
                for q in range(4):
                    m = jnp.maximum(t[lev * 4 + q], xs[q])
                    if lev < 4:
                        xs[q] = jnp.minimum(t[lev * 4 + q], xs[q])
                    t[lev * 4 + q] = m
            return tuple(t)

        return plsc.parallel_loop(0, nvec, unroll=2, carry=tops)(vec_step)

    def load_tops(i):
        return tuple(tops_buf[i + q, k] for k in range(5) for q in range(4))

    def store_tops(i, tops):
        for k in range(5):
            for q in range(4):
                tops_buf[i + q, k] = tops[k * 4 + q]

    def group_step(g, carry):
        grp = wid * _GPW + g

        for i in range(8):
            for k in range(5):
                tops_buf[i, k] = neg
        chunk_copy(grp, 0, 0, sem0).start()
        pltpu.make_async_copy(tail_hbm.at[pl.ds(grp * 8, 8)], tb.at[0], semt).start()

        def chunk_step(c, carry):
            par = lax.rem(c, 2)

            @pl.when(par == 0)
            def _():
                chunk_copy(grp, c, 0, sem0).wait()

            @pl.when(par == 1)
            def _():
                chunk_copy(grp, c, 1, sem1).wait()

            npar = lax.rem(c + 1, 2)

            @pl.when((c + 1 < _NCH) & (npar == 0))
            def _():
                chunk_copy(grp, c + 1, 0, sem0).start()

            @pl.when((c + 1 < _NCH) & (npar == 1))
            def _():
                chunk_copy(grp, c + 1, 1, sem1).start()

            for i in range(0, 8, 4):
                tops = load_tops(i)
                tops = scan_quad(cb, par, i, _CVEC, tops)
                store_tops(i, tops)
            return carry

        lax.fori_loop(0, _NCH, chunk_step, jnp.int32(0))

        pltpu.make_async_copy(tail_hbm.at[pl.ds(grp * 8, 8)], tb.at[0], semt).wait()
        for i in range(0, 8, 4):
            tops = load_tops(i)
            tops = scan_quad(tb, 0, i, _TVEC, tops)
            for k in range(5):
                for q in range(4):
                    out_buf[i + q, pl.ds(k * _L, _L)] = tops[k * 4 + q]
            for k in range(5, 8):
                for q in range(4):
                    out_buf[i + q, pl.ds(k * _L, _L)] = neg
        pltpu.sync_copy(out_buf, cand_hbm.at[pl.ds(grp * 8, 8)])
        return carry

    lax.fori_loop(0, _GPW, group_step, jnp.int32(0))


def _tc_body(cand_ref, out_ref):
    x = cand_ref[...]
    col = lax.broadcasted_iota(jnp.int32, x.shape, 1)
    s = jnp.zeros((x.shape[0], 1), jnp.float32)
    for _ in range(_TOP_K):
        m = jnp.max(x, axis=1, keepdims=True)
        idx = jnp.min(jnp.where(x == m, col, jnp.int32(_CAND)),
                      axis=1, keepdims=True)
        x = jnp.where(col == idx, jnp.float32(_NEG), x)
        s = s + m
    loss = jnp.maximum(jnp.float32(_MIN_ACT) - s * jnp.float32(1.0 / _TOP_K),
                       jnp.float32(0.0))
    out_ref[...] = jnp.reshape(jnp.sum(loss) * jnp.float32(1.0 / _R), (1, 1))


@jax.jit
def kernel(sparse_repr):
    tail = jnp.concatenate(
        [sparse_repr[:, _MAIN:],
         jnp.full((_R, _TW - (_N - _MAIN)), _NEG, jnp.float32)], axis=1)
    mesh = plsc.VectorSubcoreMesh(core_axis_name="c", subcore_axis_name="s")
    cand = pl.kernel(
        _sc_body,
        out_type=jax.ShapeDtypeStruct((_R, _CAND), jnp.float32),
        mesh=mesh,
        scratch_types=[
            pltpu.VMEM((2, 8, _CW), jnp.float32),
            pltpu.VMEM((1, 8, _TW), jnp.float32),
            pltpu.VMEM((8, 5, _L), jnp.float32),
            pltpu.VMEM((8, _CAND), jnp.float32),
            pltpu.SemaphoreType.DMA,
            pltpu.SemaphoreType.DMA,
            pltpu.SemaphoreType.DMA,
        ],
    )(sparse_repr, tail)
    loss = pl.pallas_call(
        _tc_body,
        out_shape=jax.ShapeDtypeStruct((1, 1), jnp.float32),
        in_specs=[pl.BlockSpec(memory_space=pltpu.VMEM)],
        out_specs=pl.BlockSpec(memory_space=pltpu.VMEM),
    )(cand)
    return loss[0, 0]
